# baseline (speedup 1.0000x reference)
import functools

import jax
import jax.numpy as jnp
from jax import lax
from jax.experimental import pallas as pl
from jax.experimental.pallas import tpu as pltpu
from jax.experimental.pallas import tpu_sc as plsc

_LANES = 16
_NUM_WORKERS = 32
_CHUNK = 8
_DEPTH = 3


@functools.lru_cache(maxsize=None)
def _build(B, T, D, n_batch, nw):
    b_per_w = B // nw
    t_span = b_per_w // n_batch
    n_groups = t_span // _CHUNK
    g_rows = n_batch * _CHUNK
    mesh = plsc.VectorSubcoreMesh(core_axis_name="c", subcore_axis_name="s")

    @functools.partial(
        pl.kernel,
        mesh=mesh,
        out_type=jax.ShapeDtypeStruct((B, D), jnp.float32),
        scratch_types=[
            pltpu.VMEM((b_per_w,), jnp.int32),
            pltpu.VMEM((n_groups, g_rows), jnp.int32),
            pltpu.VMEM((t_span, D), jnp.float32),
            [pltpu.VMEM((g_rows, D), jnp.float32) for _ in range(_DEPTH)],
            [pltpu.SemaphoreType.DMA for _ in range(_DEPTH)],
            [pltpu.SemaphoreType.DMA for _ in range(_DEPTH)],
            pltpu.SemaphoreType.DMA,
        ],
    )
    def sc_kernel(idx_hbm, wte_hbm, wpe_hbm, out_hbm, idx_v, oidx_v,
                  pos_v, gaths, gsems, ssems, isem):
        wid = lax.axis_index("s") * 2 + lax.axis_index("c")
        t0 = pl.multiple_of(wid * t_span, t_span)
        idx_cps = [
            pltpu.async_copy(
                idx_hbm.at[b, pl.ds(t0 + g * _CHUNK, _CHUNK)],
                idx_v.at[pl.ds(g * g_rows + b * _CHUNK, _CHUNK)], isem)
            for g in range(n_groups) for b in range(n_batch)
        ]
        pos_cp = pltpu.async_copy(wpe_hbm.at[pl.ds(t0, t_span)], pos_v, isem)
        lane = jax.lax.iota(jnp.int32, _LANES)
        for g in range(n_groups):
            for h in range(g_rows // _LANES):
                p = h * _LANES + lane
                bb = p >> 3
                rr = p & (_CHUNK - 1)
                oidx_v[g, pl.ds(h * _LANES, _LANES)] = (
                    bb * T + (t0 + g * _CHUNK) + rr)
        for cp in idx_cps:
            cp.wait()

        store_desc = [None] * _DEPTH

        def issue_gather(k):
            s = k % _DEPTH
            if store_desc[s] is not None:
                store_desc[s].wait()
                store_desc[s] = None
            return pltpu.async_copy(
                wte_hbm.at[idx_v.at[pl.ds(k * g_rows, g_rows)]],
                gaths[s], gsems[s])

        in_flight = [None] * _DEPTH
        for k in range(min(_DEPTH - 1, n_groups)):
            in_flight[k % _DEPTH] = issue_gather(k)
        pos_cp.wait()

        for j in range(n_groups):
            s = j % _DEPTH
            in_flight[s].wait()
            if j + _DEPTH - 1 < n_groups:
                in_flight[(j + _DEPTH - 1) % _DEPTH] = issue_gather(
                    j + _DEPTH - 1)

            def body(rr, carry):
                for u in range(2):
                    r = rr * 2 + u
                    for q in range(D // _LANES):
                        sl = pl.ds(q * _LANES, _LANES)
                        x = pos_v[j * _CHUNK + r, sl]
                        for b in range(n_batch):
                            row = b * _CHUNK + r
                            gaths[s][row, sl] = gaths[s][row, sl] + x
                return carry

            lax.fori_loop(0, _CHUNK // 2, body, 0)
            store_desc[s] = pltpu.async_copy(
                gaths[s], out_hbm.at[oidx_v.at[j]], ssems[s])

        for s in range(_DEPTH):
            if store_desc[s] is not None:
                store_desc[s].wait()

    return sc_kernel


def kernel(idx, wte, wpe):
    b, t = idx.shape
    v, d = wte.shape
    B = b * t
    out = _build(B, t, d, b, _NUM_WORKERS)(idx.astype(jnp.int32), wte, wpe)
    return out.reshape(b, t, d)


# speedup vs baseline: 1.0081x; 1.0081x over previous
import functools

import jax
import jax.numpy as jnp
from jax import lax
from jax.experimental import pallas as pl
from jax.experimental.pallas import tpu as pltpu
from jax.experimental.pallas import tpu_sc as plsc

_LANES = 16
_NUM_WORKERS = 32
_CHUNK = 8
_DEPTH = 3


@functools.lru_cache(maxsize=None)
def _build(B, T, D, n_batch, nw):
    b_per_w = B // nw
    t_span = b_per_w // n_batch
    n_groups = t_span // _CHUNK
    g_rows = n_batch * _CHUNK
    mesh = plsc.VectorSubcoreMesh(core_axis_name="c", subcore_axis_name="s")

    @functools.partial(
        pl.kernel,
        mesh=mesh,
        out_type=jax.ShapeDtypeStruct((B, D), jnp.float32),
        scratch_types=[
            pltpu.VMEM((b_per_w,), jnp.int32),
            pltpu.VMEM((n_groups, g_rows), jnp.int32),
            pltpu.VMEM((t_span, D), jnp.float32),
            [pltpu.VMEM((g_rows, D), jnp.float32) for _ in range(_DEPTH)],
            [pltpu.SemaphoreType.DMA for _ in range(_DEPTH)],
            [pltpu.SemaphoreType.DMA for _ in range(_DEPTH)],
            pltpu.SemaphoreType.DMA,
        ],
    )
    def sc_kernel(idx_hbm, wte_hbm, wpe_hbm, out_hbm, idx_v, oidx_v,
                  pos_v, gaths, gsems, ssems, isem):
        wid = lax.axis_index("s") * 2 + lax.axis_index("c")
        t0 = pl.multiple_of(wid * t_span, t_span)
        idx_cps = [
            pltpu.async_copy(
                idx_hbm.at[b, pl.ds(t0 + g * _CHUNK, _CHUNK)],
                idx_v.at[pl.ds(g * g_rows + b * _CHUNK, _CHUNK)], isem)
            for g in range(n_groups) for b in range(n_batch)
        ]
        pos_cp = pltpu.async_copy(wpe_hbm.at[pl.ds(t0, t_span)], pos_v, isem)
        lane = jax.lax.iota(jnp.int32, _LANES)
        for g in range(n_groups):
            for h in range(g_rows // _LANES):
                p = h * _LANES + lane
                bb = p >> 3
                rr = p & (_CHUNK - 1)
                oidx_v[g, pl.ds(h * _LANES, _LANES)] = (
                    bb * T + (t0 + g * _CHUNK) + rr)
        for cp in idx_cps:
            cp.wait()

        store_desc = [None] * _DEPTH

        def issue_gather(k):
            s = k % _DEPTH
            if store_desc[s] is not None:
                for cp in store_desc[s]:
                    cp.wait()
                store_desc[s] = None
            return pltpu.async_copy(
                wte_hbm.at[idx_v.at[pl.ds(k * g_rows, g_rows)]],
                gaths[s], gsems[s])

        in_flight = [None] * _DEPTH
        for k in range(min(_DEPTH - 1, n_groups)):
            in_flight[k % _DEPTH] = issue_gather(k)
        pos_cp.wait()

        for j in range(n_groups):
            s = j % _DEPTH
            in_flight[s].wait()
            if j + _DEPTH - 1 < n_groups:
                in_flight[(j + _DEPTH - 1) % _DEPTH] = issue_gather(
                    j + _DEPTH - 1)

            def body(r, carry):
                for q in range(D // _LANES):
                    sl = pl.ds(q * _LANES, _LANES)
                    x = pos_v[j * _CHUNK + r, sl]
                    for b in range(n_batch):
                        row = b * _CHUNK + r
                        gaths[s][row, sl] = gaths[s][row, sl] + x
                return carry

            lax.fori_loop(0, _CHUNK, body, 0)
            store_desc[s] = [
                pltpu.async_copy(
                    gaths[s].at[pl.ds(b * _CHUNK, _CHUNK)],
                    out_hbm.at[pl.ds(b * T + t0 + j * _CHUNK, _CHUNK)],
                    ssems[s])
                for b in range(n_batch)
            ]

        for s in range(_DEPTH):
            if store_desc[s] is not None:
                for cp in store_desc[s]:
                    cp.wait()

    return sc_kernel


def kernel(idx, wte, wpe):
    b, t = idx.shape
    v, d = wte.shape
    B = b * t
    out = _build(B, t, d, b, _NUM_WORKERS)(idx.astype(jnp.int32), wte, wpe)
    return out.reshape(b, t, d)
